# baseline (speedup 1.0000x reference)
import functools

import jax
import jax.numpy as jnp
from jax import lax
from jax.experimental import pallas as pl
from jax.experimental.pallas import tpu as pltpu
from jax.experimental.pallas import tpu_sc as plsc

NC, NS = 2, 16
NW = NC * NS
EB = 128
NBUF = 3
RB = 2000

_HI = lax.Precision.HIGHEST


def _dot(a, b, dims=None):
    if dims is None:
        return jnp.dot(a, b, preferred_element_type=jnp.float32, precision=_HI)
    return lax.dot_general(a, b, (dims, ((), ())),
                           preferred_element_type=jnp.float32, precision=_HI)


@functools.lru_cache(maxsize=None)
def _make_deg_kernel(n_pad, nb):
    rps = n_pad // NS
    mesh = plsc.VectorSubcoreMesh(core_axis_name="c", subcore_axis_name="s",
                                  num_cores=NC, num_subcores=NS)

    @functools.partial(
        pl.kernel,
        mesh=mesh,
        compiler_params=pltpu.CompilerParams(use_tc_tiling_on_sc=False),
        out_type=jax.ShapeDtypeStruct((NC, n_pad, 16), jnp.float32),
        scratch_types=[
            pltpu.VMEM((nb, EB), jnp.int32),
            pltpu.VMEM((EB, 16), jnp.float32),
            pltpu.VMEM_SHARED((n_pad, 16), jnp.float32),
            [pltpu.SemaphoreType.DMA for _ in range(NBUF)],
            pltpu.SemaphoreType.DMA,
        ],
    )
    def k(dst_hbm, ones_hbm, zeros_hbm, out_hbm, didx, ones_v, acc, ssems,
          zsem):
        c = lax.axis_index("c")
        s = lax.axis_index("s")
        wid = c * NS + s
        r0 = s * rps
        zd = pltpu.async_copy(zeros_hbm.at[pl.ds(r0, rps)],
                              acc.at[pl.ds(r0, rps)], zsem)
        pltpu.async_copy(ones_hbm, ones_v, ssems[0]).wait()
        pltpu.async_copy(dst_hbm.at[wid], didx, ssems[0]).wait()
        zd.wait()
        plsc.subcore_barrier()

        def wait_scatter(u):
            pltpu.make_async_copy(ones_v, acc.at[didx.at[0]], ssems[u]).wait()

        def body(p, carry):
            j = NBUF * p
            for u in range(NBUF):
                pltpu.async_copy(ones_v, acc.at[didx.at[j + u]], ssems[u],
                                 add=True)
            for u in range(NBUF):
                wait_scatter(u)
            return carry

        lax.fori_loop(0, nb // NBUF, body, 0)
        plsc.subcore_barrier()
        pltpu.async_copy(acc.at[pl.ds(r0, rps)], out_hbm.at[c, pl.ds(r0, rps)],
                         zsem).wait()

    return k


@functools.lru_cache(maxsize=None)
def _make_scatter_kernel(n_pad, nb, d, n_tab):
    rps = n_pad // NS

    mesh = plsc.VectorSubcoreMesh(core_axis_name="c", subcore_axis_name="s",
                                  num_cores=NC, num_subcores=NS)

    @functools.partial(
        pl.kernel,
        mesh=mesh,
        compiler_params=pltpu.CompilerParams(use_tc_tiling_on_sc=False),
        out_type=jax.ShapeDtypeStruct((NC, n_pad, d), jnp.float32),
        scratch_types=[
            pltpu.VMEM((nb, EB), jnp.int32),
            pltpu.VMEM((nb, EB), jnp.int32),
            [pltpu.VMEM((EB, d), jnp.float32) for _ in range(NBUF)],
            pltpu.VMEM_SHARED((n_pad, d), jnp.float32),
            pltpu.VMEM_SHARED((n_tab, d), jnp.float32),
            [pltpu.SemaphoreType.DMA for _ in range(NBUF)],
            [pltpu.SemaphoreType.DMA for _ in range(NBUF)],
            pltpu.SemaphoreType.DMA,
        ],
    )
    def k(src_hbm, dst_hbm, table_hbm, zeros_hbm, out_hbm,
          sidx, didx, rows, acc, table, gsems, ssems, zsem):
        c = lax.axis_index("c")
        s = lax.axis_index("s")
        wid = c * NS + s
        r0 = s * rps
        tps = n_tab // NS
        zd = pltpu.async_copy(zeros_hbm.at[pl.ds(r0, rps)],
                              acc.at[pl.ds(r0, rps)], zsem)
        td = pltpu.async_copy(table_hbm.at[pl.ds(s * tps, tps)],
                              table.at[pl.ds(s * tps, tps)], zsem)
        pltpu.async_copy(src_hbm.at[wid], sidx, gsems[0]).wait()
        pltpu.async_copy(dst_hbm.at[wid], didx, gsems[1]).wait()
        zd.wait()
        td.wait()
        plsc.subcore_barrier()

        def gather(j, u):
            pltpu.async_copy(table.at[sidx.at[j]], rows[u], gsems[u])

        def wait_gather(u):
            pltpu.make_async_copy(table.at[sidx.at[0]], rows[u],
                                  gsems[u]).wait()

        def scatter(j, u):
            pltpu.async_copy(rows[u], acc.at[didx.at[j]], ssems[u], add=True)

        def wait_scatter(u):
            pltpu.make_async_copy(rows[u], acc.at[didx.at[0]], ssems[u]).wait()

        for u in range(NBUF):
            gather(u, u)

        def body(p, carry):
            j = NBUF * p
            for u in range(NBUF):
                wait_gather(u)
                scatter(j + u, u)
            nxt = j + NBUF

            @pl.when(nxt < nb)
            def _():
                for u in range(NBUF):
                    wait_scatter(u)
                    gather(nxt + u, u)

            return carry

        lax.fori_loop(0, nb // NBUF, body, 0)
        for u in range(NBUF):
            wait_scatter(u)
        plsc.subcore_barrier()
        pltpu.async_copy(acc.at[pl.ds(r0, rps)], out_hbm.at[c, pl.ds(r0, rps)],
                         zsem).wait()

    return k


def _tc_first(x, w0, deg_parts):
    n, din = x.shape
    dh = w0.shape[1]
    grid = (n // RB,)

    def body(x_ref, w_ref, dp_ref, g_ref, dis_ref):
        deg = 1.0 + dp_ref[0, :, 0:1] + dp_ref[1, :, 0:1]
        dis = lax.rsqrt(deg)
        h = _dot(x_ref[...], w_ref[...])
        g_ref[...] = h * dis
        dis_ref[...] = jnp.broadcast_to(dis, (RB, dh))

    return pl.pallas_call(
        body,
        grid=grid,
        in_specs=[
            pl.BlockSpec((RB, din), lambda i: (i, 0)),
            pl.BlockSpec((din, dh), lambda i: (0, 0)),
            pl.BlockSpec((2, RB, 16), lambda i: (0, i, 0)),
        ],
        out_specs=[
            pl.BlockSpec((RB, dh), lambda i: (i, 0)),
            pl.BlockSpec((RB, dh), lambda i: (i, 0)),
        ],
        out_shape=[
            jax.ShapeDtypeStruct((n, dh), jnp.float32),
            jax.ShapeDtypeStruct((n, dh), jnp.float32),
        ],
    )(x, w0, deg_parts)


def _tc_inter(acc, g_prev, dis64, beta, lnw, lnb, w):
    n, dh = g_prev.shape

    def body(acc_ref, g_ref, dis_ref, beta_ref, lnw_ref, lnb_ref, w_ref,
             r_ref, gn_ref):
        dis = dis_ref[...]
        r = dis * (acc_ref[0] + acc_ref[1] + g_ref[...]) + beta_ref[...]
        r_ref[...] = r
        mu = jnp.mean(r, axis=-1, keepdims=True)
        cen = r - mu
        var = jnp.mean(cen * cen, axis=-1, keepdims=True)
        hn = cen * lax.rsqrt(var + 1e-5) * lnw_ref[...] + lnb_ref[...]
        h = jnp.where(hn >= 0, hn, 0.01 * hn)
        gn_ref[...] = dis * _dot(h, w_ref[...])

    return pl.pallas_call(
        body,
        grid=(n // RB,),
        in_specs=[
            pl.BlockSpec((2, RB, dh), lambda i: (0, i, 0)),
            pl.BlockSpec((RB, dh), lambda i: (i, 0)),
            pl.BlockSpec((RB, dh), lambda i: (i, 0)),
            pl.BlockSpec((1, dh), lambda i: (0, 0)),
            pl.BlockSpec((1, dh), lambda i: (0, 0)),
            pl.BlockSpec((1, dh), lambda i: (0, 0)),
            pl.BlockSpec((dh, dh), lambda i: (0, 0)),
        ],
        out_specs=[
            pl.BlockSpec((RB, dh), lambda i: (i, 0)),
            pl.BlockSpec((RB, dh), lambda i: (i, 0)),
        ],
        out_shape=[
            jax.ShapeDtypeStruct((n, dh), jnp.float32),
            jax.ShapeDtypeStruct((n, dh), jnp.float32),
        ],
    )(acc, g_prev, dis64, beta, lnw, lnb, w)


def _softmax_stats(rs, acc, g_last, dis64, beta, batch_col, t, g):
    n, dh = g_last.shape
    nr = len(rs)
    dtot = (nr + 1) * dh

    def body(*refs):
        rrefs = refs[:nr]
        (acc_ref, gl_ref, dis_ref, beta_ref, b_ref, t_ref,
         x_ref, s1_ref, cnt_ref) = refs[nr:]
        i = pl.program_id(0)
        r_last = dis_ref[...] * (acc_ref[0] + acc_ref[1] + gl_ref[...]) \
            + beta_ref[...]
        x = jnp.concatenate([r[...] for r in rrefs] + [r_last], axis=-1)
        x_ref[...] = x
        oh = (b_ref[...] == lax.broadcasted_iota(jnp.int32, (1, g), 1))
        oh = oh.astype(jnp.float32)
        s = t_ref[0, 0] * x
        p = _dot(oh, s, dims=((0,), (0,)))
        c = jnp.sum(oh, axis=0, keepdims=True)

        @pl.when(i == 0)
        def _():
            s1_ref[...] = p
            cnt_ref[...] = c

        @pl.when(i > 0)
        def _():
            s1_ref[...] += p
            cnt_ref[...] += c

    blk64 = pl.BlockSpec((RB, dh), lambda i: (i, 0))
    vec = pl.BlockSpec((1, dh), lambda i: (0, 0))
    return pl.pallas_call(
        body,
        grid=(n // RB,),
        in_specs=[blk64] * nr + [
            pl.BlockSpec((2, RB, dh), lambda i: (0, i, 0)),
            blk64, blk64, vec,
            pl.BlockSpec((RB, 1), lambda i: (i, 0)),
            pl.BlockSpec((1, 1), lambda i: (0, 0)),
        ],
        out_specs=[
            pl.BlockSpec((RB, dtot), lambda i: (i, 0)),
            pl.BlockSpec((g, dtot), lambda i: (0, 0)),
            pl.BlockSpec((1, g), lambda i: (0, 0)),
        ],
        out_shape=[
            jax.ShapeDtypeStruct((n, dtot), jnp.float32),
            jax.ShapeDtypeStruct((g, dtot), jnp.float32),
            jax.ShapeDtypeStruct((1, g), jnp.float32),
        ],
    )(*rs, acc, g_last, dis64, beta, batch_col, t)


def _softmax_final(x, batch_col, t, s1, cnt_t, g):
    n, dtot = x.shape
    nblk = n // RB

    def body(x_ref, b_ref, t_ref, s1_ref, cnt_ref, out_ref, num_s, den_s):
        i = pl.program_id(0)
        oh = (b_ref[...] == lax.broadcasted_iota(jnp.int32, (1, g), 1))
        oh = oh.astype(jnp.float32)
        shift = s1_ref[...] / jnp.maximum(cnt_ref[...], 1.0)
        p = _dot(oh, shift)
        xv = x_ref[...]
        e = jnp.exp(t_ref[0, 0] * xv - p)
        num = _dot(oh, e * xv, dims=((0,), (0,)))
        den = _dot(oh, e, dims=((0,), (0,)))

        @pl.when(i == 0)
        def _():
            num_s[...] = num
            den_s[...] = den

        @pl.when(i > 0)
        def _():
            num_s[...] += num
            den_s[...] += den

        @pl.when(i == nblk - 1)
        def _():
            d = den_s[...]
            out_ref[...] = jnp.where(d > 0, num_s[...] / d, 0.0)

    return pl.pallas_call(
        body,
        grid=(nblk,),
        in_specs=[
            pl.BlockSpec((RB, dtot), lambda i: (i, 0)),
            pl.BlockSpec((RB, 1), lambda i: (i, 0)),
            pl.BlockSpec((1, 1), lambda i: (0, 0)),
            pl.BlockSpec((g, dtot), lambda i: (0, 0)),
            pl.BlockSpec((g, 1), lambda i: (0, 0)),
        ],
        out_specs=[pl.BlockSpec((g, dtot), lambda i: (0, 0))],
        out_shape=[jax.ShapeDtypeStruct((g, dtot), jnp.float32)],
        scratch_shapes=[
            pltpu.VMEM((g, dtot), jnp.float32),
            pltpu.VMEM((g, dtot), jnp.float32),
        ],
    )(x, batch_col, t, s1, cnt_t)[0]


def kernel(node_features, edge_index, edge_type_or_attr, batch_index,
           W0, b0, ln_w, ln_b, Ws, bs, t):
    n, din = node_features.shape
    e = edge_index.shape[1]
    dh = W0.shape[1]
    nlayers = Ws.shape[0]
    g = 64

    n_pad = ((n + NS * 8) // (NS * 8)) * (NS * 8)
    epw = ((e + NW - 1) // NW + NBUF * EB - 1) // (NBUF * EB) * (NBUF * EB)
    e_pad = epw * NW
    nb = epw // EB

    src = edge_index[0]
    dst = edge_index[1]
    pad = e_pad - e
    src_p = jnp.concatenate([src, jnp.zeros((pad,), jnp.int32)])
    dst_p = jnp.concatenate([dst, jnp.full((pad,), n_pad - 1, jnp.int32)])
    src_p = src_p.reshape(NW, nb, EB)
    dst_p = dst_p.reshape(NW, nb, EB)

    zeros16 = jnp.zeros((n_pad, 16), jnp.float32)
    zeros64 = jnp.zeros((n_pad, dh), jnp.float32)
    ones_blk = jnp.ones((EB, 16), jnp.float32)

    degk = _make_deg_kernel(n_pad, nb)
    scat64 = _make_scatter_kernel(n_pad, nb, dh, n)

    deg_parts = degk(dst_p, ones_blk, zeros16)

    g_cur, dis64 = _tc_first(node_features, W0, deg_parts)

    betas = [b0.reshape(1, dh)] + [bs[i].reshape(1, dh) for i in range(nlayers)]
    results = []
    for k in range(nlayers + 1):
        acc = scat64(src_p, dst_p, g_cur, zeros64)
        if k < nlayers:
            r, g_next = _tc_inter(acc, g_cur, dis64, betas[k],
                                  ln_w[k].reshape(1, dh), ln_b[k].reshape(1, dh),
                                  Ws[k])
            results.append(r)
            g_cur = g_next

    batch_col = batch_index.reshape(n, 1)
    t2 = t.reshape(1, 1)
    node_repr, s1, cnt = _softmax_stats(results, acc, g_cur, dis64,
                                        betas[nlayers], batch_col, t2, g)
    graph_repr = _softmax_final(node_repr, batch_col, t2, s1,
                                cnt.reshape(g, 1), g)
    return (graph_repr, node_repr)


# speedup vs baseline: 1.1251x; 1.1251x over previous
import functools

import jax
import jax.numpy as jnp
from jax import lax
from jax.experimental import pallas as pl
from jax.experimental.pallas import tpu as pltpu
from jax.experimental.pallas import tpu_sc as plsc

NC, NS = 2, 16
NW = NC * NS
EB = 128
NBUF = 2
RB = 2000

_HI = lax.Precision.HIGHEST


def _dot(a, b, dims=None):
    if dims is None:
        return jnp.dot(a, b, preferred_element_type=jnp.float32, precision=_HI)
    return lax.dot_general(a, b, (dims, ((), ())),
                           preferred_element_type=jnp.float32, precision=_HI)


@functools.lru_cache(maxsize=None)
def _make_deg_kernel(n_pad, nb):
    rps = n_pad // NS
    mesh = plsc.VectorSubcoreMesh(core_axis_name="c", subcore_axis_name="s",
                                  num_cores=NC, num_subcores=NS)

    @functools.partial(
        pl.kernel,
        mesh=mesh,
        compiler_params=pltpu.CompilerParams(use_tc_tiling_on_sc=False),
        out_type=jax.ShapeDtypeStruct((NC, n_pad, 16), jnp.float32),
        scratch_types=[
            pltpu.VMEM((nb, EB), jnp.int32),
            pltpu.VMEM((EB, 16), jnp.float32),
            pltpu.VMEM_SHARED((n_pad, 16), jnp.float32),
            [pltpu.SemaphoreType.DMA for _ in range(NBUF)],
            pltpu.SemaphoreType.DMA,
        ],
    )
    def k(dst_hbm, ones_hbm, zeros_hbm, out_hbm, didx, ones_v, acc, ssems,
          zsem):
        c = lax.axis_index("c")
        s = lax.axis_index("s")
        wid = c * NS + s
        r0 = s * rps
        zd = pltpu.async_copy(zeros_hbm.at[pl.ds(r0, rps)],
                              acc.at[pl.ds(r0, rps)], zsem)
        pltpu.async_copy(ones_hbm, ones_v, ssems[0]).wait()
        pltpu.async_copy(dst_hbm.at[wid], didx, ssems[0]).wait()
        zd.wait()
        plsc.subcore_barrier()

        def wait_scatter(u):
            pltpu.make_async_copy(ones_v, acc.at[didx.at[0]], ssems[u]).wait()

        def body(p, carry):
            j = NBUF * p
            for u in range(NBUF):
                pltpu.async_copy(ones_v, acc.at[didx.at[j + u]], ssems[u],
                                 add=True)
            for u in range(NBUF):
                wait_scatter(u)
            return carry

        lax.fori_loop(0, nb // NBUF, body, 0)
        plsc.subcore_barrier()
        pltpu.async_copy(acc.at[pl.ds(r0, rps)], out_hbm.at[c, pl.ds(r0, rps)],
                         zsem).wait()

    return k


@functools.lru_cache(maxsize=None)
def _make_scatter_kernel(n_pad, nb, d, n_tab):
    rps = n_pad // NS

    mesh = plsc.VectorSubcoreMesh(core_axis_name="c", subcore_axis_name="s",
                                  num_cores=NC, num_subcores=NS)

    @functools.partial(
        pl.kernel,
        mesh=mesh,
        compiler_params=pltpu.CompilerParams(use_tc_tiling_on_sc=False),
        out_type=jax.ShapeDtypeStruct((NC, n_pad, d), jnp.float32),
        scratch_types=[
            pltpu.VMEM((nb, EB), jnp.int32),
            pltpu.VMEM((nb, EB), jnp.int32),
            [pltpu.VMEM((EB, d), jnp.float32) for _ in range(NBUF)],
            pltpu.VMEM_SHARED((n_pad, d), jnp.float32),
            pltpu.VMEM_SHARED((n_tab, d), jnp.float32),
            [pltpu.SemaphoreType.DMA for _ in range(NBUF)],
            [pltpu.SemaphoreType.DMA for _ in range(NBUF)],
            pltpu.SemaphoreType.DMA,
        ],
    )
    def k(src_hbm, dst_hbm, table_hbm, zeros_hbm, out_hbm,
          sidx, didx, rows, acc, table, gsems, ssems, zsem):
        c = lax.axis_index("c")
        s = lax.axis_index("s")
        wid = c * NS + s
        r0 = s * rps
        tps = n_tab // NS
        zd = pltpu.async_copy(zeros_hbm.at[pl.ds(r0, rps)],
                              acc.at[pl.ds(r0, rps)], zsem)
        td = pltpu.async_copy(table_hbm.at[pl.ds(s * tps, tps)],
                              table.at[pl.ds(s * tps, tps)], zsem)
        pltpu.async_copy(src_hbm.at[wid], sidx, gsems[0]).wait()
        pltpu.async_copy(dst_hbm.at[wid], didx, gsems[1]).wait()
        zd.wait()
        td.wait()
        plsc.subcore_barrier()

        def gather(j, u):
            pltpu.async_copy(table.at[sidx.at[j]], rows[u], gsems[u])

        def wait_gather(u):
            pltpu.make_async_copy(table.at[sidx.at[0]], rows[u],
                                  gsems[u]).wait()

        def scatter(j, u):
            pltpu.async_copy(rows[u], acc.at[didx.at[j]], ssems[u], add=True)

        def wait_scatter(u):
            pltpu.make_async_copy(rows[u], acc.at[didx.at[0]], ssems[u]).wait()

        for u in range(NBUF):
            gather(u, u)

        def body(p, carry):
            j = NBUF * p
            for u in range(NBUF):
                wait_gather(u)
                scatter(j + u, u)
            nxt = j + NBUF

            @pl.when(nxt < nb)
            def _():
                for u in range(NBUF):
                    wait_scatter(u)
                    gather(nxt + u, u)

            return carry

        lax.fori_loop(0, nb // NBUF, body, 0)
        for u in range(NBUF):
            wait_scatter(u)
        plsc.subcore_barrier()
        pltpu.async_copy(acc.at[pl.ds(r0, rps)], out_hbm.at[c, pl.ds(r0, rps)],
                         zsem).wait()

    return k


def _tc_first(x, w0, deg_parts):
    n, din = x.shape
    dh = w0.shape[1]
    grid = (n // RB,)

    def body(x_ref, w_ref, dp_ref, g_ref, dis_ref):
        deg = 1.0 + dp_ref[0, :, 0:1] + dp_ref[1, :, 0:1]
        dis = lax.rsqrt(deg)
        h = _dot(x_ref[...], w_ref[...])
        g_ref[...] = h * dis
        dis_ref[...] = jnp.broadcast_to(dis, (RB, dh))

    return pl.pallas_call(
        body,
        grid=grid,
        in_specs=[
            pl.BlockSpec((RB, din), lambda i: (i, 0)),
            pl.BlockSpec((din, dh), lambda i: (0, 0)),
            pl.BlockSpec((2, RB, 16), lambda i: (0, i, 0)),
        ],
        out_specs=[
            pl.BlockSpec((RB, dh), lambda i: (i, 0)),
            pl.BlockSpec((RB, dh), lambda i: (i, 0)),
        ],
        out_shape=[
            jax.ShapeDtypeStruct((n, dh), jnp.float32),
            jax.ShapeDtypeStruct((n, dh), jnp.float32),
        ],
    )(x, w0, deg_parts)


def _tc_inter(acc, g_prev, dis64, beta, lnw, lnb, w):
    n, dh = g_prev.shape

    def body(acc_ref, g_ref, dis_ref, beta_ref, lnw_ref, lnb_ref, w_ref,
             r_ref, gn_ref):
        dis = dis_ref[...]
        r = dis * (acc_ref[0] + acc_ref[1] + g_ref[...]) + beta_ref[...]
        r_ref[...] = r
        mu = jnp.mean(r, axis=-1, keepdims=True)
        cen = r - mu
        var = jnp.mean(cen * cen, axis=-1, keepdims=True)
        hn = cen * lax.rsqrt(var + 1e-5) * lnw_ref[...] + lnb_ref[...]
        h = jnp.where(hn >= 0, hn, 0.01 * hn)
        gn_ref[...] = dis * _dot(h, w_ref[...])

    return pl.pallas_call(
        body,
        grid=(n // RB,),
        in_specs=[
            pl.BlockSpec((2, RB, dh), lambda i: (0, i, 0)),
            pl.BlockSpec((RB, dh), lambda i: (i, 0)),
            pl.BlockSpec((RB, dh), lambda i: (i, 0)),
            pl.BlockSpec((1, dh), lambda i: (0, 0)),
            pl.BlockSpec((1, dh), lambda i: (0, 0)),
            pl.BlockSpec((1, dh), lambda i: (0, 0)),
            pl.BlockSpec((dh, dh), lambda i: (0, 0)),
        ],
        out_specs=[
            pl.BlockSpec((RB, dh), lambda i: (i, 0)),
            pl.BlockSpec((RB, dh), lambda i: (i, 0)),
        ],
        out_shape=[
            jax.ShapeDtypeStruct((n, dh), jnp.float32),
            jax.ShapeDtypeStruct((n, dh), jnp.float32),
        ],
    )(acc, g_prev, dis64, beta, lnw, lnb, w)


def _softmax_stats(rs, acc, g_last, dis64, beta, batch_col, t, g):
    n, dh = g_last.shape
    nr = len(rs)
    dtot = (nr + 1) * dh

    def body(*refs):
        rrefs = refs[:nr]
        (acc_ref, gl_ref, dis_ref, beta_ref, b_ref, t_ref,
         x_ref, s1_ref, cnt_ref) = refs[nr:]
        i = pl.program_id(0)
        r_last = dis_ref[...] * (acc_ref[0] + acc_ref[1] + gl_ref[...]) \
            + beta_ref[...]
        x = jnp.concatenate([r[...] for r in rrefs] + [r_last], axis=-1)
        x_ref[...] = x
        oh = (b_ref[...] == lax.broadcasted_iota(jnp.int32, (1, g), 1))
        oh = oh.astype(jnp.float32)
        s = t_ref[0, 0] * x
        p = _dot(oh, s, dims=((0,), (0,)))
        c = jnp.sum(oh, axis=0, keepdims=True)

        @pl.when(i == 0)
        def _():
            s1_ref[...] = p
            cnt_ref[...] = c

        @pl.when(i > 0)
        def _():
            s1_ref[...] += p
            cnt_ref[...] += c

    blk64 = pl.BlockSpec((RB, dh), lambda i: (i, 0))
    vec = pl.BlockSpec((1, dh), lambda i: (0, 0))
    return pl.pallas_call(
        body,
        grid=(n // RB,),
        in_specs=[blk64] * nr + [
            pl.BlockSpec((2, RB, dh), lambda i: (0, i, 0)),
            blk64, blk64, vec,
            pl.BlockSpec((RB, 1), lambda i: (i, 0)),
            pl.BlockSpec((1, 1), lambda i: (0, 0)),
        ],
        out_specs=[
            pl.BlockSpec((RB, dtot), lambda i: (i, 0)),
            pl.BlockSpec((g, dtot), lambda i: (0, 0)),
            pl.BlockSpec((1, g), lambda i: (0, 0)),
        ],
        out_shape=[
            jax.ShapeDtypeStruct((n, dtot), jnp.float32),
            jax.ShapeDtypeStruct((g, dtot), jnp.float32),
            jax.ShapeDtypeStruct((1, g), jnp.float32),
        ],
    )(*rs, acc, g_last, dis64, beta, batch_col, t)


def _softmax_final(x, batch_col, t, s1, cnt_t, g):
    n, dtot = x.shape
    nblk = n // RB

    def body(x_ref, b_ref, t_ref, s1_ref, cnt_ref, out_ref, num_s, den_s):
        i = pl.program_id(0)
        oh = (b_ref[...] == lax.broadcasted_iota(jnp.int32, (1, g), 1))
        oh = oh.astype(jnp.float32)
        shift = s1_ref[...] / jnp.maximum(cnt_ref[...], 1.0)
        p = _dot(oh, shift)
        xv = x_ref[...]
        e = jnp.exp(t_ref[0, 0] * xv - p)
        num = _dot(oh, e * xv, dims=((0,), (0,)))
        den = _dot(oh, e, dims=((0,), (0,)))

        @pl.when(i == 0)
        def _():
            num_s[...] = num
            den_s[...] = den

        @pl.when(i > 0)
        def _():
            num_s[...] += num
            den_s[...] += den

        @pl.when(i == nblk - 1)
        def _():
            d = den_s[...]
            out_ref[...] = jnp.where(d > 0, num_s[...] / d, 0.0)

    return pl.pallas_call(
        body,
        grid=(nblk,),
        in_specs=[
            pl.BlockSpec((RB, dtot), lambda i: (i, 0)),
            pl.BlockSpec((RB, 1), lambda i: (i, 0)),
            pl.BlockSpec((1, 1), lambda i: (0, 0)),
            pl.BlockSpec((g, dtot), lambda i: (0, 0)),
            pl.BlockSpec((g, 1), lambda i: (0, 0)),
        ],
        out_specs=[pl.BlockSpec((g, dtot), lambda i: (0, 0))],
        out_shape=[jax.ShapeDtypeStruct((g, dtot), jnp.float32)],
        scratch_shapes=[
            pltpu.VMEM((g, dtot), jnp.float32),
            pltpu.VMEM((g, dtot), jnp.float32),
        ],
    )(x, batch_col, t, s1, cnt_t)[0]


def kernel(node_features, edge_index, edge_type_or_attr, batch_index,
           W0, b0, ln_w, ln_b, Ws, bs, t):
    n, din = node_features.shape
    e = edge_index.shape[1]
    dh = W0.shape[1]
    nlayers = Ws.shape[0]
    g = 64

    n_pad = ((n + NS * 8) // (NS * 8)) * (NS * 8)
    epw = ((e + NW - 1) // NW + NBUF * EB - 1) // (NBUF * EB) * (NBUF * EB)
    e_pad = epw * NW
    nb = epw // EB

    src = edge_index[0]
    dst = edge_index[1]
    pad = e_pad - e
    src_p = jnp.concatenate([src, jnp.zeros((pad,), jnp.int32)])
    dst_p = jnp.concatenate([dst, jnp.full((pad,), n_pad - 1, jnp.int32)])
    src_p = src_p.reshape(NW, nb, EB)
    dst_p = dst_p.reshape(NW, nb, EB)

    zeros16 = jnp.zeros((n_pad, 16), jnp.float32)
    zeros64 = jnp.zeros((n_pad, dh), jnp.float32)
    ones_blk = jnp.ones((EB, 16), jnp.float32)

    degk = _make_deg_kernel(n_pad, nb)
    scat64 = _make_scatter_kernel(n_pad, nb, dh, n)

    deg_parts = degk(dst_p, ones_blk, zeros16)

    g_cur, dis64 = _tc_first(node_features, W0, deg_parts)

    betas = [b0.reshape(1, dh)] + [bs[i].reshape(1, dh) for i in range(nlayers)]
    results = []
    for k in range(nlayers + 1):
        acc = scat64(src_p, dst_p, g_cur, zeros64)
        if k < nlayers:
            r, g_next = _tc_inter(acc, g_cur, dis64, betas[k],
                                  ln_w[k].reshape(1, dh), ln_b[k].reshape(1, dh),
                                  Ws[k])
            results.append(r)
            g_cur = g_next

    batch_col = batch_index.reshape(n, 1)
    t2 = t.reshape(1, 1)
    node_repr, s1, cnt = _softmax_stats(results, acc, g_cur, dis64,
                                        betas[nlayers], batch_col, t2, g)
    graph_repr = _softmax_final(node_repr, batch_col, t2, s1,
                                cnt.reshape(g, 1), g)
    return (graph_repr, node_repr)


# speedup vs baseline: 1.1270x; 1.0017x over previous
import functools

import jax
import jax.numpy as jnp
from jax import lax
from jax.experimental import pallas as pl
from jax.experimental.pallas import tpu as pltpu
from jax.experimental.pallas import tpu_sc as plsc

NC, NS = 2, 16
NW = NC * NS
EB = 128
NBUF = 2
RB = 2000

_HI = lax.Precision.HIGHEST


def _dot(a, b, dims=None):
    if dims is None:
        return jnp.dot(a, b, preferred_element_type=jnp.float32, precision=_HI)
    return lax.dot_general(a, b, (dims, ((), ())),
                           preferred_element_type=jnp.float32, precision=_HI)


@functools.lru_cache(maxsize=None)
def _make_deg_kernel(n_pad, nb):
    rps = n_pad // NS
    mesh = plsc.VectorSubcoreMesh(core_axis_name="c", subcore_axis_name="s",
                                  num_cores=NC, num_subcores=NS)

    @functools.partial(
        pl.kernel,
        mesh=mesh,
        compiler_params=pltpu.CompilerParams(use_tc_tiling_on_sc=False),
        out_type=jax.ShapeDtypeStruct((NC, n_pad, 16), jnp.float32),
        scratch_types=[
            pltpu.VMEM((nb, EB), jnp.int32),
            pltpu.VMEM((EB, 16), jnp.float32),
            pltpu.VMEM_SHARED((n_pad, 16), jnp.float32),
            [pltpu.SemaphoreType.DMA for _ in range(NBUF)],
            pltpu.SemaphoreType.DMA,
        ],
    )
    def k(dst_hbm, ones_hbm, zeros_hbm, out_hbm, didx, ones_v, acc, ssems,
          zsem):
        c = lax.axis_index("c")
        s = lax.axis_index("s")
        wid = c * NS + s
        r0 = s * rps
        zd = pltpu.async_copy(zeros_hbm.at[pl.ds(r0, rps)],
                              acc.at[pl.ds(r0, rps)], zsem)
        pltpu.async_copy(ones_hbm, ones_v, ssems[0]).wait()
        pltpu.async_copy(dst_hbm.at[wid], didx, ssems[0]).wait()
        zd.wait()
        plsc.subcore_barrier()

        def wait_scatter(u):
            pltpu.make_async_copy(ones_v, acc.at[didx.at[0]], ssems[u]).wait()

        def body(p, carry):
            j = NBUF * p
            for u in range(NBUF):
                pltpu.async_copy(ones_v, acc.at[didx.at[j + u]], ssems[u],
                                 add=True)
            for u in range(NBUF):
                wait_scatter(u)
            return carry

        lax.fori_loop(0, nb // NBUF, body, 0)
        plsc.subcore_barrier()
        pltpu.async_copy(acc.at[pl.ds(r0, rps)], out_hbm.at[c, pl.ds(r0, rps)],
                         zsem).wait()

    return k


@functools.lru_cache(maxsize=None)
def _make_scatter_kernel(n_pad, nb, d, n_tab):
    rps = n_pad // NS

    mesh = plsc.VectorSubcoreMesh(core_axis_name="c", subcore_axis_name="s",
                                  num_cores=NC, num_subcores=NS)

    @functools.partial(
        pl.kernel,
        mesh=mesh,
        compiler_params=pltpu.CompilerParams(use_tc_tiling_on_sc=False),
        out_type=jax.ShapeDtypeStruct((NC, n_pad, d), jnp.float32),
        scratch_types=[
            pltpu.VMEM((nb, EB), jnp.int32),
            pltpu.VMEM((nb, EB), jnp.int32),
            [pltpu.VMEM((EB, d), jnp.float32) for _ in range(NBUF)],
            pltpu.VMEM_SHARED((n_pad, d), jnp.float32),
            pltpu.VMEM_SHARED((n_tab, d), jnp.float32),
            [pltpu.SemaphoreType.DMA for _ in range(NBUF)],
            [pltpu.SemaphoreType.DMA for _ in range(NBUF)],
            pltpu.SemaphoreType.DMA,
        ],
    )
    def k(src_hbm, dst_hbm, table_hbm, zeros_hbm, out_hbm,
          sidx, didx, rows, acc, table, gsems, ssems, zsem):
        c = lax.axis_index("c")
        s = lax.axis_index("s")
        wid = c * NS + s
        r0 = s * rps
        tps = n_tab // NS
        zd = pltpu.async_copy(zeros_hbm.at[pl.ds(r0, rps)],
                              acc.at[pl.ds(r0, rps)], zsem)
        td = pltpu.async_copy(table_hbm.at[pl.ds(s * tps, tps)],
                              table.at[pl.ds(s * tps, tps)], zsem)
        pltpu.async_copy(src_hbm.at[wid], sidx, gsems[0]).wait()
        pltpu.async_copy(dst_hbm.at[wid], didx, gsems[1]).wait()
        zd.wait()
        td.wait()
        plsc.subcore_barrier()

        def gather(j, u):
            pltpu.async_copy(table.at[sidx.at[j]], rows[u], gsems[u])

        def wait_gather(u):
            pltpu.make_async_copy(table.at[sidx.at[0]], rows[u],
                                  gsems[u]).wait()

        def scatter(j, u):
            pltpu.async_copy(rows[u], acc.at[didx.at[j]], ssems[u], add=True)

        def wait_scatter(u):
            pltpu.make_async_copy(rows[u], acc.at[didx.at[0]], ssems[u]).wait()

        for u in range(NBUF):
            gather(u, u)

        def body(p, carry):
            j = NBUF * p
            for u in range(NBUF):
                wait_gather(u)
                scatter(j + u, u)
            nxt = j + NBUF

            @pl.when(nxt < nb)
            def _():
                for u in range(NBUF):
                    wait_scatter(u)
                    gather(nxt + u, u)

            return carry

        lax.fori_loop(0, nb // NBUF, body, 0)
        for u in range(NBUF):
            wait_scatter(u)
        plsc.subcore_barrier()
        pltpu.async_copy(acc.at[pl.ds(r0, rps)], out_hbm.at[c, pl.ds(r0, rps)],
                         zsem).wait()

    return k


def _tc_mm(x, w0):
    n, din = x.shape
    dh = w0.shape[1]

    def body(x_ref, w_ref, h_ref):
        h_ref[...] = _dot(x_ref[...], w_ref[...])

    return pl.pallas_call(
        body,
        grid=(n // RB,),
        in_specs=[
            pl.BlockSpec((RB, din), lambda i: (i, 0)),
            pl.BlockSpec((din, dh), lambda i: (0, 0)),
        ],
        out_specs=[pl.BlockSpec((RB, dh), lambda i: (i, 0))],
        out_shape=[jax.ShapeDtypeStruct((n, dh), jnp.float32)],
    )(x, w0)[0]


def _tc_first(h0, deg_parts):
    n, dh = h0.shape

    def body(h_ref, dp_ref, g_ref, dis_ref):
        deg = 1.0 + dp_ref[0, :, 0:1] + dp_ref[1, :, 0:1]
        dis = lax.rsqrt(deg)
        g_ref[...] = h_ref[...] * dis
        dis_ref[...] = jnp.broadcast_to(dis, (RB, dh))

    return pl.pallas_call(
        body,
        grid=(n // RB,),
        in_specs=[
            pl.BlockSpec((RB, dh), lambda i: (i, 0)),
            pl.BlockSpec((2, RB, 16), lambda i: (0, i, 0)),
        ],
        out_specs=[
            pl.BlockSpec((RB, dh), lambda i: (i, 0)),
            pl.BlockSpec((RB, dh), lambda i: (i, 0)),
        ],
        out_shape=[
            jax.ShapeDtypeStruct((n, dh), jnp.float32),
            jax.ShapeDtypeStruct((n, dh), jnp.float32),
        ],
    )(h0, deg_parts)


def _tc_inter(acc, g_prev, dis64, beta, lnw, lnb, w):
    n, dh = g_prev.shape

    def body(acc_ref, g_ref, dis_ref, beta_ref, lnw_ref, lnb_ref, w_ref,
             r_ref, gn_ref):
        dis = dis_ref[...]
        r = dis * (acc_ref[0] + acc_ref[1] + g_ref[...]) + beta_ref[...]
        r_ref[...] = r
        mu = jnp.mean(r, axis=-1, keepdims=True)
        cen = r - mu
        var = jnp.mean(cen * cen, axis=-1, keepdims=True)
        hn = cen * lax.rsqrt(var + 1e-5) * lnw_ref[...] + lnb_ref[...]
        h = jnp.where(hn >= 0, hn, 0.01 * hn)
        gn_ref[...] = dis * _dot(h, w_ref[...])

    return pl.pallas_call(
        body,
        grid=(n // RB,),
        in_specs=[
            pl.BlockSpec((2, RB, dh), lambda i: (0, i, 0)),
            pl.BlockSpec((RB, dh), lambda i: (i, 0)),
            pl.BlockSpec((RB, dh), lambda i: (i, 0)),
            pl.BlockSpec((1, dh), lambda i: (0, 0)),
            pl.BlockSpec((1, dh), lambda i: (0, 0)),
            pl.BlockSpec((1, dh), lambda i: (0, 0)),
            pl.BlockSpec((dh, dh), lambda i: (0, 0)),
        ],
        out_specs=[
            pl.BlockSpec((RB, dh), lambda i: (i, 0)),
            pl.BlockSpec((RB, dh), lambda i: (i, 0)),
        ],
        out_shape=[
            jax.ShapeDtypeStruct((n, dh), jnp.float32),
            jax.ShapeDtypeStruct((n, dh), jnp.float32),
        ],
    )(acc, g_prev, dis64, beta, lnw, lnb, w)


def _softmax_stats(rs, acc, g_last, dis64, beta, batch_col, t, g):
    n, dh = g_last.shape
    nr = len(rs)
    dtot = (nr + 1) * dh

    def body(*refs):
        rrefs = refs[:nr]
        (acc_ref, gl_ref, dis_ref, beta_ref, b_ref, t_ref,
         x_ref, s1_ref, cnt_ref) = refs[nr:]
        i = pl.program_id(0)
        r_last = dis_ref[...] * (acc_ref[0] + acc_ref[1] + gl_ref[...]) \
            + beta_ref[...]
        x = jnp.concatenate([r[...] for r in rrefs] + [r_last], axis=-1)
        x_ref[...] = x
        oh = (b_ref[...] == lax.broadcasted_iota(jnp.int32, (1, g), 1))
        oh = oh.astype(jnp.float32)
        s = t_ref[0, 0] * x
        p = _dot(oh, s, dims=((0,), (0,)))
        c = jnp.sum(oh, axis=0, keepdims=True)

        @pl.when(i == 0)
        def _():
            s1_ref[...] = p
            cnt_ref[...] = c

        @pl.when(i > 0)
        def _():
            s1_ref[...] += p
            cnt_ref[...] += c

    blk64 = pl.BlockSpec((RB, dh), lambda i: (i, 0))
    vec = pl.BlockSpec((1, dh), lambda i: (0, 0))
    return pl.pallas_call(
        body,
        grid=(n // RB,),
        in_specs=[blk64] * nr + [
            pl.BlockSpec((2, RB, dh), lambda i: (0, i, 0)),
            blk64, blk64, vec,
            pl.BlockSpec((RB, 1), lambda i: (i, 0)),
            pl.BlockSpec((1, 1), lambda i: (0, 0)),
        ],
        out_specs=[
            pl.BlockSpec((RB, dtot), lambda i: (i, 0)),
            pl.BlockSpec((g, dtot), lambda i: (0, 0)),
            pl.BlockSpec((1, g), lambda i: (0, 0)),
        ],
        out_shape=[
            jax.ShapeDtypeStruct((n, dtot), jnp.float32),
            jax.ShapeDtypeStruct((g, dtot), jnp.float32),
            jax.ShapeDtypeStruct((1, g), jnp.float32),
        ],
    )(*rs, acc, g_last, dis64, beta, batch_col, t)


def _softmax_final(x, batch_col, t, s1, cnt_t, g):
    n, dtot = x.shape
    nblk = n // RB

    def body(x_ref, b_ref, t_ref, s1_ref, cnt_ref, out_ref, num_s, den_s):
        i = pl.program_id(0)
        oh = (b_ref[...] == lax.broadcasted_iota(jnp.int32, (1, g), 1))
        oh = oh.astype(jnp.float32)
        shift = s1_ref[...] / jnp.maximum(cnt_ref[...], 1.0)
        p = _dot(oh, shift)
        xv = x_ref[...]
        e = jnp.exp(t_ref[0, 0] * xv - p)
        num = _dot(oh, e * xv, dims=((0,), (0,)))
        den = _dot(oh, e, dims=((0,), (0,)))

        @pl.when(i == 0)
        def _():
            num_s[...] = num
            den_s[...] = den

        @pl.when(i > 0)
        def _():
            num_s[...] += num
            den_s[...] += den

        @pl.when(i == nblk - 1)
        def _():
            d = den_s[...]
            out_ref[...] = jnp.where(d > 0, num_s[...] / d, 0.0)

    return pl.pallas_call(
        body,
        grid=(nblk,),
        in_specs=[
            pl.BlockSpec((RB, dtot), lambda i: (i, 0)),
            pl.BlockSpec((RB, 1), lambda i: (i, 0)),
            pl.BlockSpec((1, 1), lambda i: (0, 0)),
            pl.BlockSpec((g, dtot), lambda i: (0, 0)),
            pl.BlockSpec((g, 1), lambda i: (0, 0)),
        ],
        out_specs=[pl.BlockSpec((g, dtot), lambda i: (0, 0))],
        out_shape=[jax.ShapeDtypeStruct((g, dtot), jnp.float32)],
        scratch_shapes=[
            pltpu.VMEM((g, dtot), jnp.float32),
            pltpu.VMEM((g, dtot), jnp.float32),
        ],
    )(x, batch_col, t, s1, cnt_t)[0]


def kernel(node_features, edge_index, edge_type_or_attr, batch_index,
           W0, b0, ln_w, ln_b, Ws, bs, t):
    n, din = node_features.shape
    e = edge_index.shape[1]
    dh = W0.shape[1]
    nlayers = Ws.shape[0]
    g = 64

    n_pad = ((n + NS * 8) // (NS * 8)) * (NS * 8)
    epw = ((e + NW - 1) // NW + NBUF * EB - 1) // (NBUF * EB) * (NBUF * EB)
    e_pad = epw * NW
    nb = epw // EB

    src = edge_index[0]
    dst = edge_index[1]
    pad = e_pad - e
    src_p = jnp.concatenate([src, jnp.zeros((pad,), jnp.int32)])
    dst_p = jnp.concatenate([dst, jnp.full((pad,), n_pad - 1, jnp.int32)])
    src_p = src_p.reshape(NW, nb, EB)
    dst_p = dst_p.reshape(NW, nb, EB)

    zeros16 = jnp.zeros((n_pad, 16), jnp.float32)
    zeros64 = jnp.zeros((n_pad, dh), jnp.float32)
    ones_blk = jnp.ones((EB, 16), jnp.float32)

    degk = _make_deg_kernel(n_pad, nb)
    scat64 = _make_scatter_kernel(n_pad, nb, dh, n)

    deg_parts = degk(dst_p, ones_blk, zeros16)
    h0 = _tc_mm(node_features, W0)

    g_cur, dis64 = _tc_first(h0, deg_parts)

    betas = [b0.reshape(1, dh)] + [bs[i].reshape(1, dh) for i in range(nlayers)]
    results = []
    for k in range(nlayers + 1):
        acc = scat64(src_p, dst_p, g_cur, zeros64)
        if k < nlayers:
            r, g_next = _tc_inter(acc, g_cur, dis64, betas[k],
                                  ln_w[k].reshape(1, dh), ln_b[k].reshape(1, dh),
                                  Ws[k])
            results.append(r)
            g_cur = g_next

    batch_col = batch_index.reshape(n, 1)
    t2 = t.reshape(1, 1)
    node_repr, s1, cnt = _softmax_stats(results, acc, g_cur, dis64,
                                        betas[nlayers], batch_col, t2, g)
    graph_repr = _softmax_final(node_repr, batch_col, t2, s1,
                                cnt.reshape(g, 1), g)
    return (graph_repr, node_repr)
